# Initial kernel scaffold; baseline (speedup 1.0000x reference)
import jax
import jax.numpy as jnp
from jax.experimental import pallas as pl


def kernel(sub_logits, original_indices):
    raise NotImplementedError("write your pallas kernel here")


# speedup vs baseline: 3.7075x; 3.7075x over previous
import functools

import jax
import jax.numpy as jnp
from jax import lax
from jax.experimental import pallas as pl
from jax.experimental.pallas import tpu as pltpu
from jax.experimental.pallas import tpu_sc as plsc

KCONST = 10.0
NSEG = 1024
TOTAL = 32768
NC = 2
NS = 16
L = 16
NW = NC * NS
CHUNK = TOTAL // NW
NVEC = CHUNK // L
SEG_PER_TILE = NSEG // NS


def _sc_segsum_body(vt_hbm, idx_hbm, out_hbm,
                    idx_v, v0, v1, v2, e0, s0, e1, s1, sh0, sh1, red, acc):
    cid = lax.axis_index("c")
    sid = lax.axis_index("s")
    wid = cid * NS + sid
    base = wid * CHUNK

    pltpu.sync_copy(idx_hbm.at[pl.ds(base, CHUNK)], idx_v.at[pl.ds(L, CHUNK)])
    neg1 = jnp.full((L,), -1, jnp.int32)
    idx_v[pl.ds(0, L)] = neg1
    idx_v[pl.ds(L + CHUNK, L)] = neg1
    pltpu.sync_copy(vt_hbm.at[pl.ds(base, CHUNK)], v0)
    pltpu.sync_copy(vt_hbm.at[pl.ds(TOTAL + base, CHUNK)], v1)
    pltpu.sync_copy(vt_hbm.at[pl.ds(2 * TOTAL + base, CHUNK)], v2)

    zf = jnp.zeros((L,), jnp.float32)

    def zero_body(i, _):
        e0[pl.ds(i * L, L)] = zf
        s0[pl.ds(i * L, L)] = zf
        e1[pl.ds(i * L, L)] = zf
        s1[pl.ds(i * L, L)] = zf
        return 0

    lax.fori_loop(0, NVEC, zero_body, 0)

    def main_body(i, carry):
        c0, c1 = carry
        off = i * L
        ids = idx_v[pl.ds(L + off, L)]
        nxt = idx_v[pl.ds(L + off + 1, L)]
        a0 = v0[pl.ds(off, L)]
        a1 = v1[pl.ds(off, L)] + v2[pl.ds(off, L)]
        p0 = plsc.cumsum(a0) + c0
        p1 = plsc.cumsum(a1) + c1
        endm = ids != nxt
        stm = jnp.logical_and(endm, nxt >= 0)
        plsc.store_scatter(e0, [ids], p0, mask=endm)
        plsc.store_scatter(e1, [ids], p1, mask=endm)
        plsc.store_scatter(s0, [nxt], p0, mask=stm)
        plsc.store_scatter(s1, [nxt], p1, mask=stm)
        return (c0 + jnp.sum(a0), c1 + jnp.sum(a1))

    lax.fori_loop(0, NVEC, main_body,
                  (jnp.float32(0.0), jnp.float32(0.0)))

    def sub_body(i, _):
        o = i * L
        e0[pl.ds(o, L)] = e0[pl.ds(o, L)] - s0[pl.ds(o, L)]
        e1[pl.ds(o, L)] = e1[pl.ds(o, L)] - s1[pl.ds(o, L)]
        return 0

    lax.fori_loop(0, NVEC, sub_body, 0)
    pltpu.sync_copy(e0, sh0.at[pl.ds(sid * NSEG, NSEG)])
    pltpu.sync_copy(e1, sh1.at[pl.ds(sid * NSEG, NSEG)])
    plsc.subcore_barrier()

    seg0 = sid * SEG_PER_TILE
    for k in range(NS):
        pltpu.sync_copy(sh0.at[pl.ds(k * NSEG + seg0, SEG_PER_TILE)],
                        red.at[pl.ds(k * SEG_PER_TILE, SEG_PER_TILE)])
    for j in range(SEG_PER_TILE // L):
        t = red[pl.ds(j * L, L)]
        for k in range(1, NS):
            t = t + red[pl.ds(k * SEG_PER_TILE + j * L, L)]
        acc[pl.ds(j * L, L)] = t
    pltpu.sync_copy(acc, out_hbm.at[pl.ds(cid * 2 * NSEG + seg0, SEG_PER_TILE)])
    for k in range(NS):
        pltpu.sync_copy(sh1.at[pl.ds(k * NSEG + seg0, SEG_PER_TILE)],
                        red.at[pl.ds(k * SEG_PER_TILE, SEG_PER_TILE)])
    for j in range(SEG_PER_TILE // L):
        t = red[pl.ds(j * L, L)]
        for k in range(1, NS):
            t = t + red[pl.ds(k * SEG_PER_TILE + j * L, L)]
        acc[pl.ds(j * L, L)] = t
    pltpu.sync_copy(acc,
                    out_hbm.at[pl.ds((cid * 2 + 1) * NSEG + seg0, SEG_PER_TILE)])


_sc_segsum = functools.partial(
    pl.kernel,
    out_type=jax.ShapeDtypeStruct((4 * NSEG,), jnp.float32),
    mesh=plsc.VectorSubcoreMesh(core_axis_name="c", subcore_axis_name="s"),
    compiler_params=pltpu.CompilerParams(needs_layout_passes=False),
    scratch_types=[
        pltpu.VMEM((2 * L + CHUNK,), jnp.int32),
        pltpu.VMEM((CHUNK,), jnp.float32),
        pltpu.VMEM((CHUNK,), jnp.float32),
        pltpu.VMEM((CHUNK,), jnp.float32),
        pltpu.VMEM((NSEG,), jnp.float32),
        pltpu.VMEM((NSEG,), jnp.float32),
        pltpu.VMEM((NSEG,), jnp.float32),
        pltpu.VMEM((NSEG,), jnp.float32),
        pltpu.VMEM_SHARED((NS * NSEG,), jnp.float32),
        pltpu.VMEM_SHARED((NS * NSEG,), jnp.float32),
        pltpu.VMEM((NS * SEG_PER_TILE,), jnp.float32),
        pltpu.VMEM((SEG_PER_TILE,), jnp.float32),
    ],
)(_sc_segsum_body)


def _tc_tail_body(x_ref, o_ref):
    x = x_ref[...].reshape(4, NSEG)
    s0 = x[0, :] + x[2, :]
    s1 = x[1, :] + x[3, :]
    p1 = 1.0 / (1.0 + jnp.exp(-KCONST * (1.0 - s1)))
    p0 = 1.0 / (1.0 + jnp.exp(-KCONST * (5.0 - s0)))
    o_ref[0, :] = jnp.log(p1 + 1e-10)
    o_ref[1, :] = jnp.log(p0 + 1e-10)


_tc_tail = pl.pallas_call(
    _tc_tail_body,
    out_shape=jax.ShapeDtypeStruct((2, NSEG), jnp.float32),
)


def kernel(sub_logits, original_indices):
    vt = sub_logits.T.reshape(-1)
    partials = _sc_segsum(vt, original_indices)
    out2 = _tc_tail(partials)
    return out2.T
